# baseline (speedup 1.0000x reference)
import functools
import jax
import jax.numpy as jnp
from jax import lax
from jax.experimental import pallas as pl
from jax.experimental.pallas import tpu as pltpu
from jax.experimental.pallas import tpu_sc as plsc

HIDDEN = 4096
EXPERTS = 64
K = 8
T_BLK = 1024

NC = 2
NS = 16
NW = NC * NS
L = 16


def _topk_body(x_ref, wt_ref, b_ref, vals_ref, idx_ref):
    x = x_ref[...]
    wt = wt_ref[...]
    logits = jax.lax.dot_general(
        x, wt, (((1,), (0,)), ((), ())),
        preferred_element_type=jnp.float32,
    ) + b_ref[...]

    fiota = jax.lax.broadcasted_iota(
        jnp.int32, (T_BLK, EXPERTS), 1).astype(jnp.float32)
    work = logits
    top_vals = []
    top_idx = []
    for _ in range(K):
        m = jnp.max(work, axis=1, keepdims=True)
        eq = work == m
        cand = jnp.where(eq, fiota, float(EXPERTS))
        sel = jnp.min(cand, axis=1, keepdims=True)
        top_vals.append(m)
        top_idx.append(sel)
        work = jnp.where(fiota == sel, -jnp.inf, work)

    vals_ref[...] = jnp.concatenate(top_vals, axis=1)
    idx_ref[...] = jnp.concatenate(
        [v.astype(jnp.int32) for v in top_idx], axis=1)


def _tc_topk(x, wt, b2, T, base_blk, n_blk):
    grid = (n_blk,)
    return pl.pallas_call(
        _topk_body,
        grid=grid,
        in_specs=[
            pl.BlockSpec((T_BLK, HIDDEN), lambda i: (i + base_blk, 0)),
            pl.BlockSpec((HIDDEN, EXPERTS), lambda i: (0, 0)),
            pl.BlockSpec((1, EXPERTS), lambda i: (0, 0)),
        ],
        out_specs=[
            pl.BlockSpec((T_BLK, K), lambda i: (i, 0)),
            pl.BlockSpec((T_BLK, K), lambda i: (i, 0)),
        ],
        out_shape=[
            jax.ShapeDtypeStruct((n_blk * T_BLK, K), jnp.float32),
            jax.ShapeDtypeStruct((n_blk * T_BLK, K), jnp.int32),
        ],
    )(x, wt, b2)


def _make_sc_scatter(T):
    tpw = T // NW
    groups = tpw // L
    mesh = plsc.VectorSubcoreMesh(core_axis_name="c", subcore_axis_name="s")

    @functools.partial(
        pl.kernel, mesh=mesh,
        out_type=jax.ShapeDtypeStruct((T, EXPERTS), jnp.float32),
        compiler_params=pltpu.CompilerParams(
            needs_layout_passes=False, use_tc_tiling_on_sc=False),
        scratch_types=[
            pltpu.VMEM((tpw, K), jnp.float32),
            pltpu.VMEM((tpw, K), jnp.int32),
            pltpu.VMEM((tpw, EXPERTS), jnp.float32),
        ],
    )
    def sc_scatter(vals_hbm, idx_hbm, out_hbm, vals_v, idx_v, out_v):
        wid = lax.axis_index("s") * NC + lax.axis_index("c")
        base = wid * tpw
        pltpu.sync_copy(vals_hbm.at[pl.ds(base, tpw)], vals_v)
        pltpu.sync_copy(idx_hbm.at[pl.ds(base, tpw)], idx_v)

        lanes = lax.iota(jnp.int32, L)
        zeros16 = jnp.zeros((L,), jnp.float32)

        def group_body(g, carry):
            row0 = g * L
            for r in range(L):
                for c in range(EXPERTS // L):
                    out_v[row0 + r, pl.ds(c * L, L)] = zeros16
            rows = row0 + lanes
            vs = [plsc.load_gather(vals_v, [rows, jnp.full((L,), j, jnp.int32)])
                  for j in range(K)]
            ids = [plsc.load_gather(idx_v, [rows, jnp.full((L,), j, jnp.int32)])
                   for j in range(K)]
            es = [jnp.exp(v - vs[0]) for v in vs]
            s = es[0]
            for e in es[1:]:
                s = s + e
            inv = 1.0 / s
            for j in range(K):
                plsc.store_scatter(out_v, [rows, ids[j]], es[j] * inv)
            return carry

        lax.fori_loop(0, groups, group_body, 0)
        pltpu.sync_copy(out_v, out_hbm.at[pl.ds(base, tpw)])

    return sc_scatter


N_CHUNKS = 2


@jax.jit
def kernel(hidden_states, W, b):
    x = hidden_states.reshape(-1, HIDDEN)
    T = x.shape[0]
    wt = W.T
    b2 = b.reshape(1, EXPERTS)
    n_blk = T // T_BLK // N_CHUNKS
    tc = T // N_CHUNKS
    sc_scatter = _make_sc_scatter(tc)
    scores_parts = []
    idx_parts = []
    for c in range(N_CHUNKS):
        top_vals, top_idx = _tc_topk(x, wt, b2, T, c * n_blk, n_blk)
        scores_parts.append(sc_scatter(top_vals, top_idx))
        idx_parts.append(top_idx)
    return (jnp.concatenate(scores_parts, axis=0),
            jnp.concatenate(idx_parts, axis=0))


# speedup vs baseline: 1.1046x; 1.1046x over previous
import functools
import jax
import jax.numpy as jnp
from jax import lax
from jax.experimental import pallas as pl
from jax.experimental.pallas import tpu as pltpu
from jax.experimental.pallas import tpu_sc as plsc

HIDDEN = 4096
EXPERTS = 64
K = 8
T_BLK = 1024

NC = 2
NS = 16
NW = NC * NS
L = 16


def _topk_body(x_ref, wt_ref, b_ref, vals_ref, idx_ref):
    x = x_ref[...]
    wt = wt_ref[...]
    logits = jax.lax.dot_general(
        x, wt, (((1,), (0,)), ((), ())),
        preferred_element_type=jnp.float32,
    ) + b_ref[...]

    fiota = jax.lax.broadcasted_iota(
        jnp.int32, (T_BLK, EXPERTS), 1).astype(jnp.float32)
    work = logits
    top_vals = []
    top_idx = []
    for _ in range(K):
        m = jnp.max(work, axis=1, keepdims=True)
        eq = work == m
        cand = jnp.where(eq, fiota, float(EXPERTS))
        sel = jnp.min(cand, axis=1, keepdims=True)
        top_vals.append(m)
        top_idx.append(sel)
        work = jnp.where(fiota == sel, -jnp.inf, work)

    vals_ref[...] = jnp.concatenate(top_vals, axis=1)
    idx_ref[...] = jnp.concatenate(
        [v.astype(jnp.int32) for v in top_idx], axis=1)


def _tc_topk(x, wt, b2, T, base_blk, n_blk):
    grid = (n_blk,)
    return pl.pallas_call(
        _topk_body,
        grid=grid,
        in_specs=[
            pl.BlockSpec((T_BLK, HIDDEN), lambda i: (i + base_blk, 0)),
            pl.BlockSpec((HIDDEN, EXPERTS), lambda i: (0, 0)),
            pl.BlockSpec((1, EXPERTS), lambda i: (0, 0)),
        ],
        out_specs=[
            pl.BlockSpec((T_BLK, K), lambda i: (i, 0)),
            pl.BlockSpec((T_BLK, K), lambda i: (i, 0)),
        ],
        out_shape=[
            jax.ShapeDtypeStruct((n_blk * T_BLK, K), jnp.float32),
            jax.ShapeDtypeStruct((n_blk * T_BLK, K), jnp.int32),
        ],
    )(x, wt, b2)


def _make_sc_scatter(T):
    tpw = T // NW
    groups = tpw // L
    mesh = plsc.VectorSubcoreMesh(core_axis_name="c", subcore_axis_name="s")

    @functools.partial(
        pl.kernel, mesh=mesh,
        out_type=jax.ShapeDtypeStruct((T, EXPERTS), jnp.float32),
        compiler_params=pltpu.CompilerParams(
            needs_layout_passes=False, use_tc_tiling_on_sc=False),
        scratch_types=[
            pltpu.VMEM((tpw, K), jnp.float32),
            pltpu.VMEM((tpw, K), jnp.int32),
            pltpu.VMEM((tpw, EXPERTS), jnp.float32),
            pltpu.SemaphoreType.DMA,
            pltpu.SemaphoreType.DMA,
        ],
    )
    def sc_scatter(vals_hbm, idx_hbm, out_hbm, vals_v, idx_v, out_v,
                   sem_v, sem_i):
        wid = lax.axis_index("s") * NC + lax.axis_index("c")
        base = wid * tpw
        cp_v = pltpu.async_copy(vals_hbm.at[pl.ds(base, tpw)], vals_v, sem_v)
        cp_i = pltpu.async_copy(idx_hbm.at[pl.ds(base, tpw)], idx_v, sem_i)

        lanes = lax.iota(jnp.int32, L)
        zeros16 = jnp.zeros((L,), jnp.float32)

        def zero_body(r, carry):
            for c in range(EXPERTS // L):
                out_v[r, pl.ds(c * L, L)] = zeros16
            return carry

        lax.fori_loop(0, tpw, zero_body, 0)
        cp_v.wait()
        cp_i.wait()

        def group_body(g, carry):
            row0 = g * L
            rows = row0 + lanes
            vs = [plsc.load_gather(vals_v, [rows, jnp.full((L,), j, jnp.int32)])
                  for j in range(K)]
            ids = [plsc.load_gather(idx_v, [rows, jnp.full((L,), j, jnp.int32)])
                   for j in range(K)]
            es = [jnp.exp(v - vs[0]) for v in vs]
            s = es[0]
            for e in es[1:]:
                s = s + e
            inv = 1.0 / s
            for j in range(K):
                plsc.store_scatter(out_v, [rows, ids[j]], es[j] * inv)
            return carry

        lax.fori_loop(0, groups, group_body, 0)
        pltpu.sync_copy(out_v, out_hbm.at[pl.ds(base, tpw)])

    return sc_scatter


N_CHUNKS = 1


@jax.jit
def kernel(hidden_states, W, b):
    x = hidden_states.reshape(-1, HIDDEN)
    T = x.shape[0]
    wt = W.T
    b2 = b.reshape(1, EXPERTS)
    n_blk = T // T_BLK // N_CHUNKS
    tc = T // N_CHUNKS
    sc_scatter = _make_sc_scatter(tc)
    scores_parts = []
    idx_parts = []
    for c in range(N_CHUNKS):
        top_vals, top_idx = _tc_topk(x, wt, b2, T, c * n_blk, n_blk)
        scores_parts.append(sc_scatter(top_vals, top_idx))
        idx_parts.append(top_idx)
    return (jnp.concatenate(scores_parts, axis=0),
            jnp.concatenate(idx_parts, axis=0))
